# Initial kernel scaffold; baseline (speedup 1.0000x reference)
import jax
import jax.numpy as jnp
from jax.experimental import pallas as pl


def kernel(elements_info, elements_mask, W_float, b_float, tm_emb):
    raise NotImplementedError("write your pallas kernel here")


# speedup vs baseline: 3.8947x; 3.8947x over previous
import jax
import jax.numpy as jnp
from jax.experimental import pallas as pl


_RB = 8192


def _body(info_ref, mask_ref, w_ref, b_ref, emb_ref, out_ref):
    x = info_ref[...]
    m = mask_ref[...]
    xm = x * m
    ff = jax.lax.dot_general(xm, w_ref[...],
                             (((1,), (0,)), ((), ())),
                             preferred_element_type=jnp.float32)
    ff = jnp.maximum(ff + b_ref[...], 0.0)
    an = xm[:, 5:6].astype(jnp.int32)
    mapped = jnp.where((an >= 21) & (an <= 30), an - 20,
                       jnp.where((an >= 39) & (an <= 48), an - 28, 0))
    onehot = (mapped == jax.lax.broadcasted_iota(jnp.int32, (1, 24), 1))
    emb = jax.lax.dot_general(onehot.astype(jnp.float32), emb_ref[...],
                              (((1,), (0,)), ((), ())),
                              preferred_element_type=jnp.float32)
    out_ref[...] = jnp.concatenate([ff, emb], axis=1) * m


def kernel(elements_info, elements_mask, W_float, b_float, tm_emb):
    B, L, C = elements_info.shape
    N = B * L
    info = elements_info.reshape(N, C)
    w7 = jnp.zeros((7, 16), jnp.float32).at[:5, :].set(W_float.T)
    mask = elements_mask.reshape(N, 1)
    emb24 = jnp.zeros((24, 8), jnp.float32).at[:21, :].set(tm_emb)
    b2 = b_float.reshape(1, 16)

    grid = (N // _RB,)
    out = pl.pallas_call(
        _body,
        grid=grid,
        in_specs=[
            pl.BlockSpec((_RB, 7), lambda i: (i, 0)),
            pl.BlockSpec((_RB, 1), lambda i: (i, 0)),
            pl.BlockSpec((7, 16), lambda i: (0, 0)),
            pl.BlockSpec((1, 16), lambda i: (0, 0)),
            pl.BlockSpec((24, 8), lambda i: (0, 0)),
        ],
        out_specs=pl.BlockSpec((_RB, 24), lambda i: (i, 0)),
        out_shape=jax.ShapeDtypeStruct((N, 24), jnp.float32),
    )(info, mask, w7, b2, emb24)
    return out.reshape(B, L, 24)
